# baseline (speedup 1.0000x reference)
import functools

import jax
import jax.numpy as jnp
from jax import lax
from jax.experimental import pallas as pl
from jax.experimental.pallas import tpu as pltpu
from jax.experimental.pallas import tpu_sc as plsc

HIDDEN = 32
LANES = 16
CHUNK = 128
PACK = 128 // HIDDEN


@functools.lru_cache(maxsize=None)
def _gmf_sc(batch):
  info = plsc.get_sparse_core_info()
  nc, ns = info.num_cores, info.num_subcores
  nw = nc * ns
  b_per_w = batch // nw
  n_chunks = b_per_w // CHUNK
  assert batch % (nw * CHUNK) == 0

  mesh = plsc.VectorSubcoreMesh(core_axis_name="c", subcore_axis_name="s")

  @functools.partial(
      pl.kernel,
      mesh=mesh,
      compiler_params=pltpu.CompilerParams(
          needs_layout_passes=False, use_tc_tiling_on_sc=True),
      out_type=jax.ShapeDtypeStruct((batch,), jnp.float32),
      scratch_types=[
          pltpu.VMEM((CHUNK,), jnp.int32),
          pltpu.VMEM((CHUNK,), jnp.int32),
          pltpu.VMEM((1, CHUNK), jnp.int32),
          pltpu.VMEM((1, CHUNK), jnp.int32),
          pltpu.VMEM((CHUNK, 128), jnp.float32),
          pltpu.VMEM((CHUNK, 128), jnp.float32),
          pltpu.VMEM((3, LANES), jnp.float32),
          pltpu.VMEM((b_per_w,), jnp.float32),
          pltpu.SemaphoreType.DMA,
          pltpu.SemaphoreType.DMA,
      ],
  )
  def k(user_hbm, food_hbm, ut_hbm, ft_hbm, wb_hbm, out_hbm,
        idx_u, idx_f, idx4_u, idx4_f, u_rows, f_rows, wb_v, out_v,
        sem_u, sem_f):
    wid = lax.axis_index("s") * nc + lax.axis_index("c")
    base = wid * b_per_w

    pltpu.sync_copy(wb_hbm, wb_v)
    w_lo = wb_v[0, :]
    w_hi = wb_v[1, :]
    b16 = wb_v[2, :]
    lane = lax.iota(jnp.int32, LANES)

    def chunk_body(c, carry):
      c0 = base + c * CHUNK
      pltpu.sync_copy(user_hbm.at[pl.ds(c0, CHUNK)], idx_u)
      pltpu.sync_copy(food_hbm.at[pl.ds(c0, CHUNK)], idx_f)

      def shift_body(it, icarry):
        r0 = it * LANES
        idx4_u[0, pl.ds(r0, LANES)] = idx_u[pl.ds(r0, LANES)] >> 2
        idx4_f[0, pl.ds(r0, LANES)] = idx_f[pl.ds(r0, LANES)] >> 2
        return icarry

      lax.fori_loop(0, CHUNK // LANES, shift_body, 0)

      cu = pltpu.async_copy(ut_hbm.at[idx4_u.at[0]], u_rows, sem_u)
      cf = pltpu.async_copy(ft_hbm.at[idx4_f.at[0]], f_rows, sem_f)
      cu.wait()
      cf.wait()

      def group_body(g, gcarry):
        i0 = g * LANES
        vu = idx_u[pl.ds(i0, LANES)]
        vf = idx_f[pl.ds(i0, LANES)]
        ou = (vu & (PACK - 1)) * HIDDEN
        of = (vf & (PACK - 1)) * HIDDEN
        acc = jnp.zeros((LANES,), jnp.float32)
        for r in range(LANES):
          i = i0 + r
          uo = ou[r]
          fo = of[r]
          uv0 = u_rows[i, pl.ds(uo, LANES)]
          uv1 = u_rows[i, pl.ds(uo + LANES, LANES)]
          fv0 = f_rows[i, pl.ds(fo, LANES)]
          fv1 = f_rows[i, pl.ds(fo + LANES, LANES)]
          p = uv0 * fv0 * w_lo + uv1 * fv1 * w_hi + b16
          s = lax.reduce_sum(p, axes=(0,))
          acc = jnp.where(lane == r, s, acc)
        out_v[pl.ds(c * CHUNK + i0, LANES)] = acc
        return gcarry

      lax.fori_loop(0, CHUNK // LANES, group_body, 0)
      return carry

    lax.fori_loop(0, n_chunks, chunk_body, 0)

    pltpu.sync_copy(out_v, out_hbm.at[pl.ds(base, b_per_w)])

  return k


def kernel(user, food, user_table, food_table, fc1_w, fc1_b):
  batch = user.shape[0]
  w = fc1_w.reshape(-1).astype(jnp.float32)
  wb = jnp.stack([
      w[:LANES],
      w[LANES:],
      jnp.broadcast_to(fc1_b.astype(jnp.float32) / LANES, (LANES,)),
  ])
  vocab = user_table.shape[0]
  ut = user_table.reshape(vocab // PACK, HIDDEN * PACK)
  ft = food_table.reshape(vocab // PACK, HIDDEN * PACK)
  return _gmf_sc(batch)(
      user.astype(jnp.int32), food.astype(jnp.int32), ut, ft, wb)


# speedup vs baseline: 1.5028x; 1.5028x over previous
import functools

import jax
import jax.numpy as jnp
from jax import lax
from jax.experimental import pallas as pl
from jax.experimental.pallas import tpu as pltpu
from jax.experimental.pallas import tpu_sc as plsc

HIDDEN = 32
LANES = 16
CHUNK = 128


@functools.lru_cache(maxsize=None)
def _gmf_sc(batch):
  info = plsc.get_sparse_core_info()
  nc, ns = info.num_cores, info.num_subcores
  nw = nc * ns
  b_per_w = batch // nw
  n_chunks = b_per_w // CHUNK
  assert batch % (nw * CHUNK) == 0

  mesh = plsc.VectorSubcoreMesh(core_axis_name="c", subcore_axis_name="s")

  @functools.partial(
      pl.kernel,
      mesh=mesh,
      compiler_params=pltpu.CompilerParams(
          needs_layout_passes=False, use_tc_tiling_on_sc=True),
      out_type=jax.ShapeDtypeStruct((batch,), jnp.float32),
      scratch_types=[
          pltpu.VMEM((CHUNK,), jnp.int32),
          pltpu.VMEM((CHUNK,), jnp.int32),
          pltpu.VMEM((CHUNK, HIDDEN), jnp.float32),
          pltpu.VMEM((CHUNK, HIDDEN), jnp.float32),
          pltpu.VMEM((3, LANES), jnp.float32),
          pltpu.VMEM((b_per_w,), jnp.float32),
          pltpu.SemaphoreType.DMA,
          pltpu.SemaphoreType.DMA,
      ],
  )
  def k(user_hbm, food_hbm, ut_hbm, ft_hbm, wb_hbm, out_hbm,
        idx_u, idx_f, u_rows, f_rows, wb_v, out_v, sem_u, sem_f):
    wid = lax.axis_index("s") * nc + lax.axis_index("c")
    base = wid * b_per_w

    pltpu.sync_copy(wb_hbm, wb_v)
    w_lo = wb_v[0, :]
    w_hi = wb_v[1, :]
    b16 = wb_v[2, :]
    lane = lax.iota(jnp.int32, LANES)

    def chunk_body(c, carry):
      c0 = base + c * CHUNK
      pltpu.sync_copy(user_hbm.at[pl.ds(c0, CHUNK)], idx_u)
      pltpu.sync_copy(food_hbm.at[pl.ds(c0, CHUNK)], idx_f)

      def issue_body(it, icarry):
        r0 = it * LANES
        vu = idx_u[pl.ds(r0, LANES)]
        vf = idx_f[pl.ds(r0, LANES)]
        for q in range(LANES):
          r = r0 + q
          pltpu.async_copy(
              ut_hbm.at[pl.ds(vu[q], 1)], u_rows.at[pl.ds(r, 1)], sem_u)
          pltpu.async_copy(
              ft_hbm.at[pl.ds(vf[q], 1)], f_rows.at[pl.ds(r, 1)], sem_f)
        return icarry

      lax.fori_loop(0, CHUNK // LANES, issue_body, 0)
      pltpu.make_async_copy(ut_hbm.at[pl.ds(0, CHUNK)], u_rows, sem_u).wait()
      pltpu.make_async_copy(ft_hbm.at[pl.ds(0, CHUNK)], f_rows, sem_f).wait()

      def group_body(g, gcarry):
        i0 = g * LANES
        acc = jnp.zeros((LANES,), jnp.float32)
        for r in range(LANES):
          i = i0 + r
          uv0 = u_rows[i, pl.ds(0, LANES)]
          uv1 = u_rows[i, pl.ds(LANES, LANES)]
          fv0 = f_rows[i, pl.ds(0, LANES)]
          fv1 = f_rows[i, pl.ds(LANES, LANES)]
          p = uv0 * fv0 * w_lo + uv1 * fv1 * w_hi + b16
          s = lax.reduce_sum(p, axes=(0,))
          acc = jnp.where(lane == r, s, acc)
        out_v[pl.ds(c * CHUNK + i0, LANES)] = acc
        return gcarry

      lax.fori_loop(0, CHUNK // LANES, group_body, 0)
      return carry

    lax.fori_loop(0, n_chunks, chunk_body, 0)

    pltpu.sync_copy(out_v, out_hbm.at[pl.ds(base, b_per_w)])

  return k


def kernel(user, food, user_table, food_table, fc1_w, fc1_b):
  batch = user.shape[0]
  w = fc1_w.reshape(-1).astype(jnp.float32)
  wb = jnp.stack([
      w[:LANES],
      w[LANES:],
      jnp.broadcast_to(fc1_b.astype(jnp.float32) / LANES, (LANES,)),
  ])
  return _gmf_sc(batch)(
      user.astype(jnp.int32), food.astype(jnp.int32),
      user_table, food_table, wb)
